# baseline (speedup 1.0000x reference)
import jax
import jax.numpy as jnp
from jax.experimental import pallas as pl


def _add_kernel(patch_ref, pos_ref, out_ref):
    out_ref[...] = patch_ref[...] + pos_ref[...]


def kernel(patch, pos_table):
    B, N, D = patch.shape
    return pl.pallas_call(
        _add_kernel,
        grid=(B,),
        in_specs=[
            pl.BlockSpec((1, N, D), lambda b: (b, 0, 0)),
            pl.BlockSpec((N, D), lambda b: (0, 0)),
        ],
        out_specs=pl.BlockSpec((1, N, D), lambda b: (b, 0, 0)),
        out_shape=jax.ShapeDtypeStruct((B, N, D), patch.dtype),
    )(patch, pos_table)
